# Initial kernel scaffold; baseline (speedup 1.0000x reference)
import jax
import jax.numpy as jnp
from jax.experimental import pallas as pl


def kernel(h, edge_index, W1, b1, W2, b2, W3, b3):
    raise NotImplementedError("write your pallas kernel here")


# speedup vs baseline: 3.4010x; 3.4010x over previous
import functools

import jax
import jax.numpy as jnp
from jax import lax
from jax.experimental import pallas as pl
from jax.experimental.pallas import tpu as pltpu
from jax.experimental.pallas import tpu_sc as plsc

N = 10000
NP = 10112
NC = 2
NS = 16
NW = NC * NS
ZR = NP // NS
TRASH = N + 100
E = 320000
EPT = 10240
CH = 64
NCH = EPT // CH
W1W = 144
W2W = 128
R = 2528


def _sc_agg_body(w, x_hbm, src_hbm, dst_hbm, z_hbm, out_hbm,
                 acc, sidx, didx, rows0, rows1, sem0, sem1):
    c = lax.axis_index("c")
    s = lax.axis_index("s")
    wid = c * NS + s

    pltpu.sync_copy(z_hbm, acc.at[pl.ds(s * ZR, ZR)])

    pltpu.sync_copy(src_hbm.at[wid], sidx)
    pltpu.sync_copy(dst_hbm.at[wid], didx)
    plsc.subcore_barrier()

    rows = (rows0, rows1)
    sems = (sem0, sem1)
    pltpu.async_copy(x_hbm.at[sidx.at[0]], rows0, sem0)

    def blk(b, carry):
        for j in range(2):
            ch = b * 2 + j
            nxt = ch + 1
            pltpu.make_async_copy(x_hbm.at[sidx.at[ch]], rows[j], sems[j]).wait()

            @pl.when(nxt < NCH)
            def _():
                pltpu.async_copy(x_hbm.at[sidx.at[nxt]], rows[1 - j], sems[1 - j])

            pltpu.sync_copy(rows[j], acc.at[didx.at[ch]], add=True)
        return carry

    lax.fori_loop(0, NCH // 2, blk, 0)

    plsc.subcore_barrier()
    pltpu.sync_copy(acc.at[pl.ds(s * ZR, ZR)], out_hbm.at[c, pl.ds(s * ZR, ZR)])


def _make_sc_agg(w):
    mesh = plsc.VectorSubcoreMesh(core_axis_name="c", subcore_axis_name="s")
    return pl.kernel(
        functools.partial(_sc_agg_body, w),
        out_type=jax.ShapeDtypeStruct((NC, NP, w), jnp.float32),
        mesh=mesh,
        scratch_types=[
            pltpu.VMEM_SHARED((NP, w), jnp.float32),
            pltpu.VMEM((NCH, CH), jnp.int32),
            pltpu.VMEM((NCH, CH), jnp.int32),
            pltpu.VMEM((CH, w), jnp.float32),
            pltpu.VMEM((CH, w), jnp.float32),
            pltpu.SemaphoreType.DMA,
            pltpu.SemaphoreType.DMA,
        ],
        compiler_params=pltpu.CompilerParams(use_tc_tiling_on_sc=False),
        name=f"sage_sc_agg_{w}",
    )


def _tc1_body(sp_ref, x_ref, wa_ref, wb_ref, b_ref, o_ref, invd_ref):
    v = sp_ref[...]
    ssum = v[0, :, :128] + v[1, :, :128]
    deg = v[0, :, 128:129] + v[1, :, 128:129]
    invd = 1.0 / jnp.maximum(deg, 1.0)
    mean = ssum * invd
    y = (jnp.dot(x_ref[...], wa_ref[...], preferred_element_type=jnp.float32)
         + jnp.dot(mean, wb_ref[...], preferred_element_type=jnp.float32)
         + b_ref[...])
    o_ref[...] = jnp.maximum(y, 0.0)
    invd_ref[...] = jnp.broadcast_to(invd, (invd.shape[0], 128))


def _tc23_body(relu, sp_ref, x_ref, invd_ref, wa_ref, wb_ref, b_ref, o_ref):
    v = sp_ref[...]
    mean = (v[0] + v[1]) * invd_ref[...]
    y = (jnp.dot(x_ref[...], wa_ref[...], preferred_element_type=jnp.float32)
         + jnp.dot(mean, wb_ref[...], preferred_element_type=jnp.float32)
         + b_ref[...])
    o_ref[...] = jnp.maximum(y, 0.0) if relu else y


def _tc1(spair, x, wa, wb, b):
    return pl.pallas_call(
        _tc1_body,
        grid=(NP // R,),
        in_specs=[
            pl.BlockSpec((NC, R, W1W), lambda i: (0, i, 0)),
            pl.BlockSpec((R, 128), lambda i: (i, 0)),
            pl.BlockSpec((128, 128), lambda i: (0, 0)),
            pl.BlockSpec((128, 128), lambda i: (0, 0)),
            pl.BlockSpec((1, 128), lambda i: (0, 0)),
        ],
        out_specs=[
            pl.BlockSpec((R, 128), lambda i: (i, 0)),
            pl.BlockSpec((R, 128), lambda i: (i, 0)),
        ],
        out_shape=[
            jax.ShapeDtypeStruct((NP, 128), jnp.float32),
            jax.ShapeDtypeStruct((NP, 128), jnp.float32),
        ],
        name="sage_tc1",
    )(spair, x, wa, wb, b)


def _tc23(spair, x, invd, wa, wb, b, relu):
    return pl.pallas_call(
        functools.partial(_tc23_body, relu),
        grid=(NP // R,),
        in_specs=[
            pl.BlockSpec((NC, R, W2W), lambda i: (0, i, 0)),
            pl.BlockSpec((R, 128), lambda i: (i, 0)),
            pl.BlockSpec((R, 128), lambda i: (i, 0)),
            pl.BlockSpec((128, 128), lambda i: (0, 0)),
            pl.BlockSpec((128, 128), lambda i: (0, 0)),
            pl.BlockSpec((1, 128), lambda i: (0, 0)),
        ],
        out_specs=pl.BlockSpec((R, 128), lambda i: (i, 0)),
        out_shape=jax.ShapeDtypeStruct((NP, 128), jnp.float32),
        name="sage_tc23",
    )(spair, x, invd, wa, wb, b)


def kernel(h, edge_index, W1, b1, W2, b2, W3, b3):
    f32 = jnp.float32
    src = edge_index[0]
    dst = edge_index[1]
    pad = NW * EPT - E
    src_p = jnp.concatenate([src, jnp.zeros((pad,), jnp.int32)]).reshape(NW, NCH, CH)
    dst_p = jnp.concatenate([dst, jnp.full((pad,), TRASH, jnp.int32)]).reshape(NW, NCH, CH)

    hp = jnp.zeros((NP, W1W), f32)
    hp = hp.at[:N, :128].set(h)
    hp = hp.at[:, 128].set(1.0)
    x0 = hp[:, :128]

    z1 = jnp.zeros((ZR, W1W), f32)
    z2 = jnp.zeros((ZR, W2W), f32)

    agg1 = _make_sc_agg(W1W)
    agg2 = _make_sc_agg(W2W)

    sp1 = agg1(hp, src_p, dst_p, z1)
    x1, invd = _tc1(sp1, x0, W1[:128], W1[128:], b1.reshape(1, 128))
    sp2 = agg2(x1, src_p, dst_p, z2)
    x2 = _tc23(sp2, x1, invd, W2[:128], W2[128:], b2.reshape(1, 128), True)
    sp3 = agg2(x2, src_p, dst_p, z2)
    x3 = _tc23(sp3, x2, invd, W3[:128], W3[128:], b3.reshape(1, 128), False)
    return x3[:N]


# speedup vs baseline: 6.7253x; 1.9774x over previous
import functools

import jax
import jax.numpy as jnp
from jax import lax
from jax.experimental import pallas as pl
from jax.experimental.pallas import tpu as pltpu
from jax.experimental.pallas import tpu_sc as plsc

N = 10000
NP = 10016
NC = 2
NS = 16
ZR = NP // NS
TRASH = N + 8
E = 320000
EPT = 20480
CH = 128
NCH = EPT // CH
KB = 8
NB = NCH // KB
W1H = 80
W2H = 64
R = 2504


def _sc_agg_body(w, xs_hbm, src_hbm, dst_hbm, z_hbm, out_hbm,
                 xtab, acc, sidx, didx, rows0, rows1, sem0, sem1):
    c = lax.axis_index("c")
    s = lax.axis_index("s")

    pltpu.sync_copy(xs_hbm.at[c, pl.ds(s * ZR, ZR)], xtab.at[pl.ds(s * ZR, ZR)])
    pltpu.sync_copy(z_hbm, acc.at[pl.ds(s * ZR, ZR)])
    plsc.subcore_barrier()

    rows = (rows0, rows1)
    sems = (sem0, sem1)

    def blk(b, carry):
        pltpu.sync_copy(src_hbm.at[s, pl.ds(b * KB, KB)], sidx)
        pltpu.sync_copy(dst_hbm.at[s, pl.ds(b * KB, KB)], didx)
        pltpu.async_copy(xtab.at[sidx.at[0]], rows0, sem0)
        for ch in range(KB):
            j = ch % 2
            pltpu.make_async_copy(xtab.at[sidx.at[ch]], rows[j], sems[j]).wait()
            if ch + 1 < KB:
                pltpu.async_copy(xtab.at[sidx.at[ch + 1]], rows[1 - j], sems[1 - j])
            pltpu.sync_copy(rows[j], acc.at[didx.at[ch]], add=True)
        return carry

    lax.fori_loop(0, NB, blk, 0)

    plsc.subcore_barrier()
    pltpu.sync_copy(acc.at[pl.ds(s * ZR, ZR)], out_hbm.at[c, pl.ds(s * ZR, ZR)])


def _make_sc_agg(w):
    mesh = plsc.VectorSubcoreMesh(core_axis_name="c", subcore_axis_name="s")
    return pl.kernel(
        _sc_agg_body_w(w),
        out_type=jax.ShapeDtypeStruct((NC, NP, w), jnp.float32),
        mesh=mesh,
        scratch_types=[
            pltpu.VMEM_SHARED((NP, w), jnp.float32),
            pltpu.VMEM_SHARED((NP, w), jnp.float32),
            pltpu.VMEM((KB, CH), jnp.int32),
            pltpu.VMEM((KB, CH), jnp.int32),
            pltpu.VMEM((CH, w), jnp.float32),
            pltpu.VMEM((CH, w), jnp.float32),
            pltpu.SemaphoreType.DMA,
            pltpu.SemaphoreType.DMA,
        ],
        compiler_params=pltpu.CompilerParams(use_tc_tiling_on_sc=False),
        name=f"sage_sc_agg_{w}",
    )


def _sc_agg_body_w(w):
    return functools.partial(_sc_agg_body, w)


def _tc1_body(sp_ref, x_ref, wa_ref, wb_ref, b_ref, o_ref, os_ref, invd_ref):
    v = sp_ref[...]
    ssum = jnp.concatenate([v[0, :, :W2H], v[1, :, :W2H]], axis=1)
    deg = v[0, :, W2H:W2H + 1]
    invd = 1.0 / jnp.maximum(deg, 1.0)
    mean = ssum * invd
    y = (jnp.dot(x_ref[...], wa_ref[...], preferred_element_type=jnp.float32)
         + jnp.dot(mean, wb_ref[...], preferred_element_type=jnp.float32)
         + b_ref[...])
    y = jnp.maximum(y, 0.0)
    o_ref[...] = y
    os_ref[0] = y[:, :W2H]
    os_ref[1] = y[:, W2H:]
    invd_ref[...] = jnp.broadcast_to(invd, (invd.shape[0], 128))


def _tc23_body(relu, sp_ref, x_ref, invd_ref, wa_ref, wb_ref, b_ref, o_ref, os_ref):
    v = sp_ref[...]
    ssum = jnp.concatenate([v[0], v[1]], axis=1)
    mean = ssum * invd_ref[...]
    y = (jnp.dot(x_ref[...], wa_ref[...], preferred_element_type=jnp.float32)
         + jnp.dot(mean, wb_ref[...], preferred_element_type=jnp.float32)
         + b_ref[...])
    if relu:
        y = jnp.maximum(y, 0.0)
    o_ref[...] = y
    os_ref[0] = y[:, :W2H]
    os_ref[1] = y[:, W2H:]


def _tc1(spair, x, wa, wb, b):
    return pl.pallas_call(
        _tc1_body,
        grid=(NP // R,),
        in_specs=[
            pl.BlockSpec((NC, R, W1H), lambda i: (0, i, 0)),
            pl.BlockSpec((R, 128), lambda i: (i, 0)),
            pl.BlockSpec((128, 128), lambda i: (0, 0)),
            pl.BlockSpec((128, 128), lambda i: (0, 0)),
            pl.BlockSpec((1, 128), lambda i: (0, 0)),
        ],
        out_specs=[
            pl.BlockSpec((R, 128), lambda i: (i, 0)),
            pl.BlockSpec((NC, R, W2H), lambda i: (0, i, 0)),
            pl.BlockSpec((R, 128), lambda i: (i, 0)),
        ],
        out_shape=[
            jax.ShapeDtypeStruct((NP, 128), jnp.float32),
            jax.ShapeDtypeStruct((NC, NP, W2H), jnp.float32),
            jax.ShapeDtypeStruct((NP, 128), jnp.float32),
        ],
        name="sage_tc1",
    )(spair, x, wa, wb, b)


def _tc23(spair, x, invd, wa, wb, b, relu, split_out):
    out_specs = [pl.BlockSpec((R, 128), lambda i: (i, 0))]
    out_shape = [jax.ShapeDtypeStruct((NP, 128), jnp.float32)]
    if split_out:
        out_specs.append(pl.BlockSpec((NC, R, W2H), lambda i: (0, i, 0)))
        out_shape.append(jax.ShapeDtypeStruct((NC, NP, W2H), jnp.float32))
        body = functools.partial(_tc23_body, relu)
    else:
        def body(sp_ref, x_ref, invd_ref, wa_ref, wb_ref, b_ref, o_ref):
            class _Null:
                def __setitem__(self, k, v):
                    pass
            _tc23_body(relu, sp_ref, x_ref, invd_ref, wa_ref, wb_ref, b_ref,
                       o_ref, _Null())
    return pl.pallas_call(
        body,
        grid=(NP // R,),
        in_specs=[
            pl.BlockSpec((NC, R, W2H), lambda i: (0, i, 0)),
            pl.BlockSpec((R, 128), lambda i: (i, 0)),
            pl.BlockSpec((R, 128), lambda i: (i, 0)),
            pl.BlockSpec((128, 128), lambda i: (0, 0)),
            pl.BlockSpec((128, 128), lambda i: (0, 0)),
            pl.BlockSpec((1, 128), lambda i: (0, 0)),
        ],
        out_specs=out_specs if len(out_specs) > 1 else out_specs[0],
        out_shape=out_shape if len(out_shape) > 1 else out_shape[0],
        name="sage_tc23",
    )(spair, x, invd, wa, wb, b)


def kernel(h, edge_index, W1, b1, W2, b2, W3, b3):
    f32 = jnp.float32
    src = edge_index[0]
    dst = edge_index[1]
    pad = NS * EPT - E
    src_p = jnp.concatenate([src, jnp.zeros((pad,), jnp.int32)]).reshape(NS, NCH, CH)
    dst_p = jnp.concatenate([dst, jnp.full((pad,), TRASH, jnp.int32)]).reshape(NS, NCH, CH)

    hs = jnp.zeros((NC, NP, W1H), f32)
    hs = hs.at[0, :N, :W2H].set(h[:, :W2H])
    hs = hs.at[1, :N, :W2H].set(h[:, W2H:])
    hs = hs.at[:, :, W2H].set(1.0)
    x0 = jnp.zeros((NP, 128), f32).at[:N].set(h)

    z1 = jnp.zeros((ZR, W1H), f32)
    z2 = jnp.zeros((ZR, W2H), f32)

    agg1 = _make_sc_agg(W1H)
    agg2 = _make_sc_agg(W2H)

    sp1 = agg1(hs, src_p, dst_p, z1)
    x1, x1s, invd = _tc1(sp1, x0, W1[:128], W1[128:], b1.reshape(1, 128))
    sp2 = agg2(x1s, src_p, dst_p, z2)
    x2, x2s = _tc23(sp2, x1, invd, W2[:128], W2[128:], b2.reshape(1, 128), True, True)
    sp3 = agg2(x2s, src_p, dst_p, z2)
    x3 = _tc23(sp3, x2, invd, W3[:128], W3[128:], b3.reshape(1, 128), False, False)
    return x3[:N]


# speedup vs baseline: 7.8555x; 1.1680x over previous
import functools

import jax
import jax.numpy as jnp
from jax import lax
from jax.experimental import pallas as pl
from jax.experimental.pallas import tpu as pltpu
from jax.experimental.pallas import tpu_sc as plsc

N = 10000
NP = 10016
NC = 2
NS = 16
ZR = NP // NS
E = 320000
CH = 128
ER = E // CH
CPT = ER // NS
KB = 12
NB = CPT // KB
W1H = 80
W2H = 64
R = 2504


def _sc_agg_body(w, xs_hbm, src_hbm, dst_hbm, z_hbm, out_hbm,
                 xtab, acc, sidx, didx, rows0, rows1, sem0, sem1):
    c = lax.axis_index("c")
    s = lax.axis_index("s")

    pltpu.sync_copy(xs_hbm.at[c, pl.ds(s * ZR, ZR)], xtab.at[pl.ds(s * ZR, ZR)])
    pltpu.sync_copy(z_hbm, acc.at[pl.ds(s * ZR, ZR)])
    plsc.subcore_barrier()

    rows = (rows0, rows1)
    sems = (sem0, sem1)
    base = s * CPT

    def blk(b, carry):
        pltpu.sync_copy(src_hbm.at[pl.ds(base + b * KB, KB)], sidx)
        pltpu.sync_copy(dst_hbm.at[pl.ds(base + b * KB, KB)], didx)
        pltpu.async_copy(xtab.at[sidx.at[0]], rows0, sem0)
        for ch in range(KB):
            j = ch % 2
            pltpu.make_async_copy(xtab.at[sidx.at[ch]], rows[j], sems[j]).wait()
            if ch + 1 < KB:
                pltpu.async_copy(xtab.at[sidx.at[ch + 1]], rows[1 - j], sems[1 - j])
            pltpu.sync_copy(rows[j], acc.at[didx.at[ch]], add=True)
        return carry

    lax.fori_loop(0, NB, blk, 0)

    @pl.when(s < ER - NS * CPT)
    def _():
        pltpu.sync_copy(src_hbm.at[pl.ds(NS * CPT + s, 1)], sidx.at[pl.ds(0, 1)])
        pltpu.sync_copy(dst_hbm.at[pl.ds(NS * CPT + s, 1)], didx.at[pl.ds(0, 1)])
        pltpu.async_copy(xtab.at[sidx.at[0]], rows0, sem0).wait()
        pltpu.sync_copy(rows0, acc.at[didx.at[0]], add=True)

    plsc.subcore_barrier()
    pltpu.sync_copy(acc.at[pl.ds(s * ZR, ZR)], out_hbm.at[c, pl.ds(s * ZR, ZR)])


def _make_sc_agg(w):
    mesh = plsc.VectorSubcoreMesh(core_axis_name="c", subcore_axis_name="s")
    return pl.kernel(
        functools.partial(_sc_agg_body, w),
        out_type=jax.ShapeDtypeStruct((NC, NP, w), jnp.float32),
        mesh=mesh,
        scratch_types=[
            pltpu.VMEM_SHARED((NP, w), jnp.float32),
            pltpu.VMEM_SHARED((NP, w), jnp.float32),
            pltpu.VMEM((KB, CH), jnp.int32),
            pltpu.VMEM((KB, CH), jnp.int32),
            pltpu.VMEM((CH, w), jnp.float32),
            pltpu.VMEM((CH, w), jnp.float32),
            pltpu.SemaphoreType.DMA,
            pltpu.SemaphoreType.DMA,
        ],
        compiler_params=pltpu.CompilerParams(use_tc_tiling_on_sc=False),
        name=f"sage_sc_agg_{w}",
    )


def _prolog_body(h_ref, hs_ref, x0_ref):
    hb = h_ref[...]
    x0_ref[...] = hb
    hs_ref[0, :, :W2H] = hb[:, :W2H]
    hs_ref[1, :, :W2H] = hb[:, W2H:]
    hs_ref[:, :, W2H:W2H + 1] = jnp.ones((NC, hb.shape[0], 1), jnp.float32)
    hs_ref[:, :, W2H + 1:] = jnp.zeros((NC, hb.shape[0], W1H - W2H - 1), jnp.float32)


def _prolog(h):
    return pl.pallas_call(
        _prolog_body,
        grid=(NP // R,),
        in_specs=[pl.BlockSpec((R, 128), lambda i: (i, 0))],
        out_specs=[
            pl.BlockSpec((NC, R, W1H), lambda i: (0, i, 0)),
            pl.BlockSpec((R, 128), lambda i: (i, 0)),
        ],
        out_shape=[
            jax.ShapeDtypeStruct((NC, NP, W1H), jnp.float32),
            jax.ShapeDtypeStruct((NP, 128), jnp.float32),
        ],
        name="sage_prolog",
    )(h)


def _tc1_body(sp_ref, x_ref, wa_ref, wb_ref, b_ref, o_ref, os_ref, invd_ref):
    v = sp_ref[...]
    ssum = jnp.concatenate([v[0, :, :W2H], v[1, :, :W2H]], axis=1)
    deg = v[0, :, W2H:W2H + 1]
    invd = 1.0 / jnp.maximum(deg, 1.0)
    mean = ssum * invd
    y = (jnp.dot(x_ref[...], wa_ref[...], preferred_element_type=jnp.float32)
         + jnp.dot(mean, wb_ref[...], preferred_element_type=jnp.float32)
         + b_ref[...])
    y = jnp.maximum(y, 0.0)
    o_ref[...] = y
    os_ref[0] = y[:, :W2H]
    os_ref[1] = y[:, W2H:]
    invd_ref[...] = jnp.broadcast_to(invd, (invd.shape[0], 128))


def _tc23_body(relu, sp_ref, x_ref, invd_ref, wa_ref, wb_ref, b_ref, o_ref, os_ref):
    v = sp_ref[...]
    ssum = jnp.concatenate([v[0], v[1]], axis=1)
    mean = ssum * invd_ref[...]
    y = (jnp.dot(x_ref[...], wa_ref[...], preferred_element_type=jnp.float32)
         + jnp.dot(mean, wb_ref[...], preferred_element_type=jnp.float32)
         + b_ref[...])
    if relu:
        y = jnp.maximum(y, 0.0)
    o_ref[...] = y
    if os_ref is not None:
        os_ref[0] = y[:, :W2H]
        os_ref[1] = y[:, W2H:]


def _tc1(spair, x, wa, wb, b):
    return pl.pallas_call(
        _tc1_body,
        grid=(NP // R,),
        in_specs=[
            pl.BlockSpec((NC, R, W1H), lambda i: (0, i, 0)),
            pl.BlockSpec((R, 128), lambda i: (i, 0)),
            pl.BlockSpec((128, 128), lambda i: (0, 0)),
            pl.BlockSpec((128, 128), lambda i: (0, 0)),
            pl.BlockSpec((1, 128), lambda i: (0, 0)),
        ],
        out_specs=[
            pl.BlockSpec((R, 128), lambda i: (i, 0)),
            pl.BlockSpec((NC, R, W2H), lambda i: (0, i, 0)),
            pl.BlockSpec((R, 128), lambda i: (i, 0)),
        ],
        out_shape=[
            jax.ShapeDtypeStruct((NP, 128), jnp.float32),
            jax.ShapeDtypeStruct((NC, NP, W2H), jnp.float32),
            jax.ShapeDtypeStruct((NP, 128), jnp.float32),
        ],
        name="sage_tc1",
    )(spair, x, wa, wb, b)


def _tc23(spair, x, invd, wa, wb, b, relu, split_out):
    if split_out:
        body = functools.partial(_tc23_body, relu)
        out_specs = [
            pl.BlockSpec((R, 128), lambda i: (i, 0)),
            pl.BlockSpec((NC, R, W2H), lambda i: (0, i, 0)),
        ]
        out_shape = [
            jax.ShapeDtypeStruct((NP, 128), jnp.float32),
            jax.ShapeDtypeStruct((NC, NP, W2H), jnp.float32),
        ]
    else:
        def body(sp_ref, x_ref, invd_ref, wa_ref, wb_ref, b_ref, o_ref):
            _tc23_body(relu, sp_ref, x_ref, invd_ref, wa_ref, wb_ref, b_ref,
                       o_ref, None)
        out_specs = pl.BlockSpec((R, 128), lambda i: (i, 0))
        out_shape = jax.ShapeDtypeStruct((NP, 128), jnp.float32)
    return pl.pallas_call(
        body,
        grid=(NP // R,),
        in_specs=[
            pl.BlockSpec((NC, R, W2H), lambda i: (0, i, 0)),
            pl.BlockSpec((R, 128), lambda i: (i, 0)),
            pl.BlockSpec((R, 128), lambda i: (i, 0)),
            pl.BlockSpec((128, 128), lambda i: (0, 0)),
            pl.BlockSpec((128, 128), lambda i: (0, 0)),
            pl.BlockSpec((1, 128), lambda i: (0, 0)),
        ],
        out_specs=out_specs,
        out_shape=out_shape,
        name="sage_tc23",
    )(spair, x, invd, wa, wb, b)


def kernel(h, edge_index, W1, b1, W2, b2, W3, b3):
    f32 = jnp.float32
    src_r = edge_index[0].reshape(ER, CH)
    dst_r = edge_index[1].reshape(ER, CH)

    z1 = jnp.zeros((ZR, W1H), f32)
    z2 = jnp.zeros((ZR, W2H), f32)

    agg1 = _make_sc_agg(W1H)
    agg2 = _make_sc_agg(W2H)

    hs, x0 = _prolog(h)
    sp1 = agg1(hs, src_r, dst_r, z1)
    x1, x1s, invd = _tc1(sp1, x0, W1[:128], W1[128:], b1.reshape(1, 128))
    sp2 = agg2(x1s, src_r, dst_r, z2)
    x2, x2s = _tc23(sp2, x1, invd, W2[:128], W2[128:], b2.reshape(1, 128), True, True)
    sp3 = agg2(x2s, src_r, dst_r, z2)
    x3 = _tc23(sp3, x2, invd, W3[:128], W3[128:], b3.reshape(1, 128), False, False)
    return x3[:N]


# speedup vs baseline: 9.2254x; 1.1744x over previous
import functools

import jax
import jax.numpy as jnp
from jax import lax
from jax.experimental import pallas as pl
from jax.experimental.pallas import tpu as pltpu
from jax.experimental.pallas import tpu_sc as plsc

N = 10000
NP = 10016
NC = 2
NS = 16
ZR = NP // NS
E = 320000
CH = 128
ER = E // CH
CPT = ER // NS
KB = 26
NB = CPT // KB
W1H = 80
W2H = 64
R = 2504


def _sc_agg1_body(x_hbm, ones_hbm, src_hbm, dst_hbm, z_hbm, out0_hbm, out1_hbm,
                  xtab, acc, sidx, didx, rows0, rows1, sem0, sem1):
    c = lax.axis_index("c")
    s = lax.axis_index("s")
    rs = pl.ds(s * ZR, ZR)

    pltpu.sync_copy(x_hbm.at[rs, pl.ds(c * W2H, W2H)], xtab.at[rs, pl.ds(0, W2H)])
    pltpu.sync_copy(ones_hbm.at[rs, pl.ds(0, W1H - W2H)],
                    xtab.at[rs, pl.ds(W2H, W1H - W2H)])
    pltpu.sync_copy(z_hbm, acc.at[rs])
    plsc.subcore_barrier()

    _edge_loop(W1H, src_hbm, dst_hbm, xtab, acc, sidx, didx,
               rows0, rows1, sem0, sem1, s)

    plsc.subcore_barrier()

    @pl.when(c == 0)
    def _():
        pltpu.sync_copy(acc.at[rs], out0_hbm.at[rs, pl.ds(0, W1H)])

    @pl.when(c == 1)
    def _():
        pltpu.sync_copy(acc.at[rs], out1_hbm.at[rs, pl.ds(0, W1H)])


def _sc_agg2_body(x_hbm, src_hbm, dst_hbm, z_hbm, out_hbm,
                  xtab, acc, sidx, didx, rows0, rows1, sem0, sem1):
    c = lax.axis_index("c")
    s = lax.axis_index("s")
    rs = pl.ds(s * ZR, ZR)
    cs = pl.ds(c * W2H, W2H)

    pltpu.sync_copy(x_hbm.at[rs, cs], xtab.at[rs])
    pltpu.sync_copy(z_hbm, acc.at[rs])
    plsc.subcore_barrier()

    _edge_loop(W2H, src_hbm, dst_hbm, xtab, acc, sidx, didx,
               rows0, rows1, sem0, sem1, s)

    plsc.subcore_barrier()
    pltpu.sync_copy(acc.at[rs], out_hbm.at[rs, cs])


def _edge_loop(w, src_hbm, dst_hbm, xtab, acc, sidx, didx,
               rows0, rows1, sem0, sem1, s):
    rows = (rows0, rows1)
    sems = (sem0, sem1)
    base = s * CPT

    def blk(b, carry):
        pltpu.sync_copy(src_hbm.at[pl.ds(base + b * KB, KB)], sidx)
        pltpu.sync_copy(dst_hbm.at[pl.ds(base + b * KB, KB)], didx)
        pltpu.async_copy(xtab.at[sidx.at[0]], rows0, sem0)
        for ch in range(KB):
            j = ch % 2
            pltpu.make_async_copy(xtab.at[sidx.at[ch]], rows[j], sems[j]).wait()
            if ch + 1 < KB:
                pltpu.async_copy(xtab.at[sidx.at[ch + 1]], rows[1 - j], sems[1 - j])
            pltpu.sync_copy(rows[j], acc.at[didx.at[ch]], add=True)
        return carry

    lax.fori_loop(0, NB, blk, 0)

    @pl.when(s < ER - NS * CPT)
    def _():
        pltpu.sync_copy(src_hbm.at[pl.ds(NS * CPT + s, 1)], sidx.at[pl.ds(0, 1)])
        pltpu.sync_copy(dst_hbm.at[pl.ds(NS * CPT + s, 1)], didx.at[pl.ds(0, 1)])
        pltpu.async_copy(xtab.at[sidx.at[0]], rows0, sem0).wait()
        pltpu.sync_copy(rows0, acc.at[didx.at[0]], add=True)


def _sc_scratch(w):
    return [
        pltpu.VMEM_SHARED((NP, w), jnp.float32),
        pltpu.VMEM_SHARED((NP, w), jnp.float32),
        pltpu.VMEM((KB, CH), jnp.int32),
        pltpu.VMEM((KB, CH), jnp.int32),
        pltpu.VMEM((CH, w), jnp.float32),
        pltpu.VMEM((CH, w), jnp.float32),
        pltpu.SemaphoreType.DMA,
        pltpu.SemaphoreType.DMA,
    ]


_SC_MESH = plsc.VectorSubcoreMesh(core_axis_name="c", subcore_axis_name="s")
_SC_PARAMS = pltpu.CompilerParams(use_tc_tiling_on_sc=False)

_agg1 = pl.kernel(
    _sc_agg1_body,
    out_type=[jax.ShapeDtypeStruct((NP, 128), jnp.float32),
              jax.ShapeDtypeStruct((NP, 128), jnp.float32)],
    mesh=_SC_MESH,
    scratch_types=_sc_scratch(W1H),
    compiler_params=_SC_PARAMS,
    name="sage_sc_agg1",
)

_agg2 = pl.kernel(
    _sc_agg2_body,
    out_type=jax.ShapeDtypeStruct((NP, 128), jnp.float32),
    mesh=_SC_MESH,
    scratch_types=_sc_scratch(W2H),
    compiler_params=_SC_PARAMS,
    name="sage_sc_agg2",
)


def _prolog_body(h_ref, x0_ref, ones_ref):
    x0_ref[...] = h_ref[...]
    ones_ref[...] = jnp.ones_like(ones_ref)


def _prolog(h):
    return pl.pallas_call(
        _prolog_body,
        grid=(NP // R,),
        in_specs=[pl.BlockSpec((R, 128), lambda i: (i, 0))],
        out_specs=[
            pl.BlockSpec((R, 128), lambda i: (i, 0)),
            pl.BlockSpec((R, 128), lambda i: (i, 0)),
        ],
        out_shape=[
            jax.ShapeDtypeStruct((NP, 128), jnp.float32),
            jax.ShapeDtypeStruct((NP, 128), jnp.float32),
        ],
        name="sage_prolog",
    )(h)


def _tc1_body(sp0_ref, sp1_ref, x_ref, wa_ref, wb_ref, b_ref, o_ref, invd_ref):
    v0 = sp0_ref[...]
    v1 = sp1_ref[...]
    ssum = jnp.concatenate([v0[:, :W2H], v1[:, :W2H]], axis=1)
    deg = v0[:, W2H:W2H + 1]
    invd = 1.0 / jnp.maximum(deg, 1.0)
    mean = ssum * invd
    y = (jnp.dot(x_ref[...], wa_ref[...], preferred_element_type=jnp.float32)
         + jnp.dot(mean, wb_ref[...], preferred_element_type=jnp.float32)
         + b_ref[...])
    o_ref[...] = jnp.maximum(y, 0.0)
    invd_ref[...] = jnp.broadcast_to(invd, (invd.shape[0], 128))


def _tc23_body(relu, sp_ref, x_ref, invd_ref, wa_ref, wb_ref, b_ref, o_ref):
    mean = sp_ref[...] * invd_ref[...]
    y = (jnp.dot(x_ref[...], wa_ref[...], preferred_element_type=jnp.float32)
         + jnp.dot(mean, wb_ref[...], preferred_element_type=jnp.float32)
         + b_ref[...])
    if relu:
        y = jnp.maximum(y, 0.0)
    o_ref[...] = y


_MAT_SPEC = pl.BlockSpec((128, 128), lambda i: (0, 0))
_VEC_SPEC = pl.BlockSpec((1, 128), lambda i: (0, 0))
_ROW_SPEC = pl.BlockSpec((R, 128), lambda i: (i, 0))


def _tc1(sp0, sp1, x, wa, wb, b):
    return pl.pallas_call(
        _tc1_body,
        grid=(NP // R,),
        in_specs=[_ROW_SPEC, _ROW_SPEC, _ROW_SPEC, _MAT_SPEC, _MAT_SPEC, _VEC_SPEC],
        out_specs=[_ROW_SPEC, _ROW_SPEC],
        out_shape=[
            jax.ShapeDtypeStruct((NP, 128), jnp.float32),
            jax.ShapeDtypeStruct((NP, 128), jnp.float32),
        ],
        name="sage_tc1",
    )(sp0, sp1, x, wa, wb, b)


def _tc23(sp, x, invd, wa, wb, b, relu):
    return pl.pallas_call(
        functools.partial(_tc23_body, relu),
        grid=(NP // R,),
        in_specs=[_ROW_SPEC, _ROW_SPEC, _ROW_SPEC, _MAT_SPEC, _MAT_SPEC, _VEC_SPEC],
        out_specs=_ROW_SPEC,
        out_shape=jax.ShapeDtypeStruct((NP, 128), jnp.float32),
        name="sage_tc23",
    )(sp, x, invd, wa, wb, b)


def kernel(h, edge_index, W1, b1, W2, b2, W3, b3):
    f32 = jnp.float32
    src_r = edge_index[0].reshape(ER, CH)
    dst_r = edge_index[1].reshape(ER, CH)

    z1 = jnp.zeros((ZR, W1H), f32)
    z2 = jnp.zeros((ZR, W2H), f32)

    x0, ones = _prolog(h)
    sp0, sp1 = _agg1(x0, ones, src_r, dst_r, z1)
    x1, invd = _tc1(sp0, sp1, x0, W1[:128], W1[128:], b1.reshape(1, 128))
    s2 = _agg2(x1, src_r, dst_r, z2)
    x2 = _tc23(s2, x1, invd, W2[:128], W2[128:], b2.reshape(1, 128), True)
    s3 = _agg2(x2, src_r, dst_r, z2)
    x3 = _tc23(s3, x2, invd, W3[:128], W3[128:], b3.reshape(1, 128), False)
    return x3[:N]


# speedup vs baseline: 9.6189x; 1.0427x over previous
import functools

import jax
import jax.numpy as jnp
from jax import lax
from jax.experimental import pallas as pl
from jax.experimental.pallas import tpu as pltpu
from jax.experimental.pallas import tpu_sc as plsc

N = 10000
NC = 2
NS = 16
ZR = N // NS
E = 320000
CH = 128
ER = E // CH
CPT = ER // NS
KB = 26
NB = CPT // KB
W1H = 80
W2H = 64
R = 2000


def _sc_agg1_body(x_hbm, ones_hbm, src_hbm, dst_hbm, z_hbm, out0_hbm, out1_hbm,
                  xtab, acc, sidx, didx, rows0, rows1, sem0, sem1):
    c = lax.axis_index("c")
    s = lax.axis_index("s")
    rs = pl.ds(s * ZR, ZR)

    pltpu.sync_copy(x_hbm.at[rs, pl.ds(c * W2H, W2H)], xtab.at[rs, pl.ds(0, W2H)])
    pltpu.sync_copy(ones_hbm.at[:, pl.ds(0, W1H - W2H)],
                    xtab.at[rs, pl.ds(W2H, W1H - W2H)])
    pltpu.sync_copy(z_hbm.at[:, pl.ds(0, W1H)], acc.at[rs])
    plsc.subcore_barrier()

    _edge_loop(src_hbm, dst_hbm, xtab, acc, sidx, didx,
               rows0, rows1, sem0, sem1, s)

    plsc.subcore_barrier()

    @pl.when(c == 0)
    def _():
        pltpu.sync_copy(acc.at[rs], out0_hbm.at[rs, pl.ds(0, W1H)])

    @pl.when(c == 1)
    def _():
        pltpu.sync_copy(acc.at[rs], out1_hbm.at[rs, pl.ds(0, W1H)])


def _sc_agg2_body(x_hbm, src_hbm, dst_hbm, z_hbm, out_hbm,
                  xtab, acc, sidx0, didx0, sidx1, didx1,
                  rows0, rows1, sem0, sem1, semi):
    c = lax.axis_index("c")
    s = lax.axis_index("s")
    rs = pl.ds(s * ZR, ZR)
    cs = pl.ds(c * W2H, W2H)

    pltpu.sync_copy(x_hbm.at[rs, cs], xtab.at[rs])
    pltpu.sync_copy(z_hbm.at[:, pl.ds(0, W2H)], acc.at[rs])
    plsc.subcore_barrier()

    rows = (rows0, rows1)
    sems = (sem0, sem1)
    sidxs = (sidx0, sidx1)
    didxs = (didx0, didx1)
    base = s * CPT

    for p in range(2):
        pltpu.async_copy(src_hbm.at[pl.ds(base + p * KB, KB)], sidxs[p], semi)
        pltpu.async_copy(dst_hbm.at[pl.ds(base + p * KB, KB)], didxs[p], semi)

    def blk2(b2, carry):
        for p in range(2):
            bb = b2 * 2 + p
            off = pl.ds(base + bb * KB, KB)
            pltpu.make_async_copy(src_hbm.at[off], sidxs[p], semi).wait()
            pltpu.make_async_copy(dst_hbm.at[off], didxs[p], semi).wait()
            sidx = sidxs[p]
            didx = didxs[p]
            pltpu.async_copy(xtab.at[sidx.at[0]], rows0, sem0)
            for ch in range(KB):
                j = ch % 2
                pltpu.make_async_copy(xtab.at[sidx.at[ch]], rows[j], sems[j]).wait()
                if ch + 1 < KB:
                    pltpu.async_copy(xtab.at[sidx.at[ch + 1]], rows[1 - j],
                                     sems[1 - j])
                pltpu.sync_copy(rows[j], acc.at[didx.at[ch]], add=True)

            @pl.when(bb + 2 < NB)
            def _():
                off2 = pl.ds(base + (bb + 2) * KB, KB)
                pltpu.async_copy(src_hbm.at[off2], sidxs[p], semi)
                pltpu.async_copy(dst_hbm.at[off2], didxs[p], semi)
        return carry

    lax.fori_loop(0, NB // 2, blk2, 0)

    @pl.when(s < ER - NS * CPT)
    def _():
        pltpu.sync_copy(src_hbm.at[pl.ds(NS * CPT + s, 1)], sidx0.at[pl.ds(0, 1)])
        pltpu.sync_copy(dst_hbm.at[pl.ds(NS * CPT + s, 1)], didx0.at[pl.ds(0, 1)])
        pltpu.async_copy(xtab.at[sidx0.at[0]], rows0, sem0).wait()
        pltpu.sync_copy(rows0, acc.at[didx0.at[0]], add=True)

    plsc.subcore_barrier()
    pltpu.sync_copy(acc.at[rs], out_hbm.at[rs, cs])


def _edge_loop(src_hbm, dst_hbm, xtab, acc, sidx, didx,
               rows0, rows1, sem0, sem1, s):
    rows = (rows0, rows1)
    sems = (sem0, sem1)
    base = s * CPT

    def blk(b, carry):
        pltpu.sync_copy(src_hbm.at[pl.ds(base + b * KB, KB)], sidx)
        pltpu.sync_copy(dst_hbm.at[pl.ds(base + b * KB, KB)], didx)
        pltpu.async_copy(xtab.at[sidx.at[0]], rows0, sem0)
        for ch in range(KB):
            j = ch % 2
            pltpu.make_async_copy(xtab.at[sidx.at[ch]], rows[j], sems[j]).wait()
            if ch + 1 < KB:
                pltpu.async_copy(xtab.at[sidx.at[ch + 1]], rows[1 - j], sems[1 - j])
            pltpu.sync_copy(rows[j], acc.at[didx.at[ch]], add=True)
        return carry

    lax.fori_loop(0, NB, blk, 0)

    @pl.when(s < ER - NS * CPT)
    def _():
        pltpu.sync_copy(src_hbm.at[pl.ds(NS * CPT + s, 1)], sidx.at[pl.ds(0, 1)])
        pltpu.sync_copy(dst_hbm.at[pl.ds(NS * CPT + s, 1)], didx.at[pl.ds(0, 1)])
        pltpu.async_copy(xtab.at[sidx.at[0]], rows0, sem0).wait()
        pltpu.sync_copy(rows0, acc.at[didx.at[0]], add=True)


_SC_MESH = plsc.VectorSubcoreMesh(core_axis_name="c", subcore_axis_name="s")
_SC_PARAMS = pltpu.CompilerParams(use_tc_tiling_on_sc=False)

_agg1 = pl.kernel(
    _sc_agg1_body,
    out_type=[jax.ShapeDtypeStruct((N, 128), jnp.float32),
              jax.ShapeDtypeStruct((N, 128), jnp.float32)],
    mesh=_SC_MESH,
    scratch_types=[
        pltpu.VMEM_SHARED((N, W1H), jnp.float32),
        pltpu.VMEM_SHARED((N, W1H), jnp.float32),
        pltpu.VMEM((KB, CH), jnp.int32),
        pltpu.VMEM((KB, CH), jnp.int32),
        pltpu.VMEM((CH, W1H), jnp.float32),
        pltpu.VMEM((CH, W1H), jnp.float32),
        pltpu.SemaphoreType.DMA,
        pltpu.SemaphoreType.DMA,
    ],
    compiler_params=_SC_PARAMS,
    name="sage_sc_agg1",
)

_agg2 = pl.kernel(
    _sc_agg2_body,
    out_type=jax.ShapeDtypeStruct((N, 128), jnp.float32),
    mesh=_SC_MESH,
    scratch_types=[
        pltpu.VMEM_SHARED((N, W2H), jnp.float32),
        pltpu.VMEM_SHARED((N, W2H), jnp.float32),
        pltpu.VMEM((KB, CH), jnp.int32),
        pltpu.VMEM((KB, CH), jnp.int32),
        pltpu.VMEM((KB, CH), jnp.int32),
        pltpu.VMEM((KB, CH), jnp.int32),
        pltpu.VMEM((CH, W2H), jnp.float32),
        pltpu.VMEM((CH, W2H), jnp.float32),
        pltpu.SemaphoreType.DMA,
        pltpu.SemaphoreType.DMA,
        pltpu.SemaphoreType.DMA,
    ],
    compiler_params=_SC_PARAMS,
    name="sage_sc_agg2",
)


def _tc1_body(sp0_ref, sp1_ref, x_ref, wa_ref, wb_ref, b_ref, o_ref, invd_ref):
    v0 = sp0_ref[...]
    v1 = sp1_ref[...]
    ssum = jnp.concatenate([v0[:, :W2H], v1[:, :W2H]], axis=1)
    deg = v0[:, W2H:W2H + 1]
    invd = 1.0 / jnp.maximum(deg, 1.0)
    mean = ssum * invd
    y = (jnp.dot(x_ref[...], wa_ref[...], preferred_element_type=jnp.float32)
         + jnp.dot(mean, wb_ref[...], preferred_element_type=jnp.float32)
         + b_ref[...])
    o_ref[...] = jnp.maximum(y, 0.0)
    invd_ref[...] = jnp.broadcast_to(invd, (invd.shape[0], 128))


def _tc23_body(relu, sp_ref, x_ref, invd_ref, wa_ref, wb_ref, b_ref, o_ref):
    mean = sp_ref[...] * invd_ref[...]
    y = (jnp.dot(x_ref[...], wa_ref[...], preferred_element_type=jnp.float32)
         + jnp.dot(mean, wb_ref[...], preferred_element_type=jnp.float32)
         + b_ref[...])
    if relu:
        y = jnp.maximum(y, 0.0)
    o_ref[...] = y


_MAT_SPEC = pl.BlockSpec((128, 128), lambda i: (0, 0))
_VEC_SPEC = pl.BlockSpec((1, 128), lambda i: (0, 0))
_ROW_SPEC = pl.BlockSpec((R, 128), lambda i: (i, 0))


def _tc1(sp0, sp1, x, wa, wb, b):
    return pl.pallas_call(
        _tc1_body,
        grid=(N // R,),
        in_specs=[_ROW_SPEC, _ROW_SPEC, _ROW_SPEC, _MAT_SPEC, _MAT_SPEC, _VEC_SPEC],
        out_specs=[_ROW_SPEC, _ROW_SPEC],
        out_shape=[
            jax.ShapeDtypeStruct((N, 128), jnp.float32),
            jax.ShapeDtypeStruct((N, 128), jnp.float32),
        ],
        name="sage_tc1",
    )(sp0, sp1, x, wa, wb, b)


def _tc23(sp, x, invd, wa, wb, b, relu):
    return pl.pallas_call(
        functools.partial(_tc23_body, relu),
        grid=(N // R,),
        in_specs=[_ROW_SPEC, _ROW_SPEC, _ROW_SPEC, _MAT_SPEC, _MAT_SPEC, _VEC_SPEC],
        out_specs=_ROW_SPEC,
        out_shape=jax.ShapeDtypeStruct((N, 128), jnp.float32),
        name="sage_tc23",
    )(sp, x, invd, wa, wb, b)


def kernel(h, edge_index, W1, b1, W2, b2, W3, b3):
    f32 = jnp.float32
    src_r = edge_index[0].reshape(ER, CH)
    dst_r = edge_index[1].reshape(ER, CH)

    ones = jnp.ones((ZR, 128), f32)
    z = jnp.zeros((ZR, 128), f32)

    sp0, sp1 = _agg1(h, ones, src_r, dst_r, z)
    x1, invd = _tc1(sp0, sp1, h, W1[:128], W1[128:], b1.reshape(1, 128))
    s2 = _agg2(x1, src_r, dst_r, z)
    x2 = _tc23(s2, x1, invd, W2[:128], W2[128:], b2.reshape(1, 128), True)
    s3 = _agg2(x2, src_r, dst_r, z)
    x3 = _tc23(s3, x2, invd, W3[:128], W3[128:], b3.reshape(1, 128), False)
    return x3


# speedup vs baseline: 9.8925x; 1.0284x over previous
import functools

import jax
import jax.numpy as jnp
from jax import lax
from jax.experimental import pallas as pl
from jax.experimental.pallas import tpu as pltpu
from jax.experimental.pallas import tpu_sc as plsc

N = 10000
NC = 2
NS = 16
ZR = N // NS
E = 320000
CH = 128
ER = E // CH
CPT = ER // NS
KB = 26
NB = CPT // KB
DW = 16
HW = 64
R = 2000


def _edge_loop(src_hbm, dst_hbm, xtab, acc, sidxs, didxs,
               rows, sems, semi, s, deg=None):
    base = s * CPT

    for p in range(2):
        pltpu.async_copy(src_hbm.at[pl.ds(base + p * KB, KB)], sidxs[p], semi)
        pltpu.async_copy(dst_hbm.at[pl.ds(base + p * KB, KB)], didxs[p], semi)

    def blk2(b2, carry):
        for p in range(2):
            bb = b2 * 2 + p
            off = pl.ds(base + bb * KB, KB)
            pltpu.make_async_copy(src_hbm.at[off], sidxs[p], semi).wait()
            pltpu.make_async_copy(dst_hbm.at[off], didxs[p], semi).wait()
            sidx = sidxs[p]
            didx = didxs[p]
            pltpu.async_copy(xtab.at[sidx.at[0]], rows[0], sems[0])
            for ch in range(KB):
                j = ch % 2
                pltpu.make_async_copy(xtab.at[sidx.at[ch]], rows[j], sems[j]).wait()
                if ch + 1 < KB:
                    pltpu.async_copy(xtab.at[sidx.at[ch + 1]], rows[1 - j],
                                     sems[1 - j])
                pltpu.sync_copy(rows[j], acc.at[didx.at[ch]], add=True)
                if deg is not None:
                    c, orows, dacc = deg

                    @pl.when((bb < NB // 2) == (c == 0))
                    def _():
                        pltpu.sync_copy(orows, dacc.at[didx.at[ch]], add=True)

            @pl.when(bb + 2 < NB)
            def _():
                off2 = pl.ds(base + (bb + 2) * KB, KB)
                pltpu.async_copy(src_hbm.at[off2], sidxs[p], semi)
                pltpu.async_copy(dst_hbm.at[off2], didxs[p], semi)
        return carry

    lax.fori_loop(0, NB // 2, blk2, 0)

    @pl.when(s < ER - NS * CPT)
    def _():
        pltpu.sync_copy(src_hbm.at[pl.ds(NS * CPT + s, 1)], sidxs[0].at[pl.ds(0, 1)])
        pltpu.sync_copy(dst_hbm.at[pl.ds(NS * CPT + s, 1)], didxs[0].at[pl.ds(0, 1)])
        pltpu.async_copy(xtab.at[sidxs[0].at[0]], rows[0], sems[0]).wait()
        pltpu.sync_copy(rows[0], acc.at[didxs[0].at[0]], add=True)
        if deg is not None:
            c, orows, dacc = deg

            @pl.when(c == 0)
            def _():
                pltpu.sync_copy(orows, dacc.at[didxs[0].at[0]], add=True)


def _sc_agg1_body(x_hbm, ones_hbm, src_hbm, dst_hbm, z_hbm, out0_hbm, out1_hbm,
                  xtab, acc, dacc, orows, sidx0, didx0, sidx1, didx1,
                  rows0, rows1, sem0, sem1, semi):
    c = lax.axis_index("c")
    s = lax.axis_index("s")
    rs = pl.ds(s * ZR, ZR)

    pltpu.sync_copy(x_hbm.at[rs, pl.ds(c * HW, HW)], xtab.at[rs])
    pltpu.sync_copy(z_hbm.at[:, pl.ds(0, HW)], acc.at[rs])
    pltpu.sync_copy(z_hbm.at[:, pl.ds(0, DW)], dacc.at[rs])
    pltpu.sync_copy(ones_hbm.at[pl.ds(0, CH), pl.ds(0, DW)], orows)
    plsc.subcore_barrier()

    _edge_loop(src_hbm, dst_hbm, xtab, acc, (sidx0, sidx1), (didx0, didx1),
               (rows0, rows1), (sem0, sem1), semi, s, deg=(c, orows, dacc))

    plsc.subcore_barrier()

    @pl.when(c == 0)
    def _():
        pltpu.sync_copy(acc.at[rs], out0_hbm.at[rs, pl.ds(0, HW)])
        pltpu.sync_copy(dacc.at[rs], out0_hbm.at[rs, pl.ds(HW, DW)])

    @pl.when(c == 1)
    def _():
        pltpu.sync_copy(acc.at[rs], out1_hbm.at[rs, pl.ds(0, HW)])
        pltpu.sync_copy(dacc.at[rs], out1_hbm.at[rs, pl.ds(HW, DW)])


def _sc_agg2_body(x_hbm, src_hbm, dst_hbm, z_hbm, out_hbm,
                  xtab, acc, sidx0, didx0, sidx1, didx1,
                  rows0, rows1, sem0, sem1, semi):
    c = lax.axis_index("c")
    s = lax.axis_index("s")
    rs = pl.ds(s * ZR, ZR)
    cs = pl.ds(c * HW, HW)

    pltpu.sync_copy(x_hbm.at[rs, cs], xtab.at[rs])
    pltpu.sync_copy(z_hbm.at[:, pl.ds(0, HW)], acc.at[rs])
    plsc.subcore_barrier()

    _edge_loop(src_hbm, dst_hbm, xtab, acc, (sidx0, sidx1), (didx0, didx1),
               (rows0, rows1), (sem0, sem1), semi, s)

    plsc.subcore_barrier()
    pltpu.sync_copy(acc.at[rs], out_hbm.at[rs, cs])


_SC_MESH = plsc.VectorSubcoreMesh(core_axis_name="c", subcore_axis_name="s")
_SC_PARAMS = pltpu.CompilerParams(use_tc_tiling_on_sc=False)


def _idx_rows_scratch():
    return [
        pltpu.VMEM((KB, CH), jnp.int32),
        pltpu.VMEM((KB, CH), jnp.int32),
        pltpu.VMEM((KB, CH), jnp.int32),
        pltpu.VMEM((KB, CH), jnp.int32),
        pltpu.VMEM((CH, HW), jnp.float32),
        pltpu.VMEM((CH, HW), jnp.float32),
        pltpu.SemaphoreType.DMA,
        pltpu.SemaphoreType.DMA,
        pltpu.SemaphoreType.DMA,
    ]


_agg1 = pl.kernel(
    _sc_agg1_body,
    out_type=[jax.ShapeDtypeStruct((N, 128), jnp.float32),
              jax.ShapeDtypeStruct((N, 128), jnp.float32)],
    mesh=_SC_MESH,
    scratch_types=[
        pltpu.VMEM_SHARED((N, HW), jnp.float32),
        pltpu.VMEM_SHARED((N, HW), jnp.float32),
        pltpu.VMEM_SHARED((N, DW), jnp.float32),
        pltpu.VMEM((CH, DW), jnp.float32),
    ] + _idx_rows_scratch(),
    compiler_params=_SC_PARAMS,
    name="sage_sc_agg1",
)

_agg2 = pl.kernel(
    _sc_agg2_body,
    out_type=jax.ShapeDtypeStruct((N, 128), jnp.float32),
    mesh=_SC_MESH,
    scratch_types=[
        pltpu.VMEM_SHARED((N, HW), jnp.float32),
        pltpu.VMEM_SHARED((N, HW), jnp.float32),
    ] + _idx_rows_scratch(),
    compiler_params=_SC_PARAMS,
    name="sage_sc_agg2",
)


def _tc1_body(sp0_ref, sp1_ref, x_ref, wa_ref, wb_ref, b_ref, o_ref, invd_ref):
    v0 = sp0_ref[...]
    v1 = sp1_ref[...]
    ssum = jnp.concatenate([v0[:, :HW], v1[:, :HW]], axis=1)
    deg = v0[:, HW:HW + 1] + v1[:, HW:HW + 1]
    invd = 1.0 / jnp.maximum(deg, 1.0)
    mean = ssum * invd
    y = (jnp.dot(x_ref[...], wa_ref[...], preferred_element_type=jnp.float32)
         + jnp.dot(mean, wb_ref[...], preferred_element_type=jnp.float32)
         + b_ref[...])
    o_ref[...] = jnp.maximum(y, 0.0)
    invd_ref[...] = jnp.broadcast_to(invd, (invd.shape[0], 128))


def _tc23_body(relu, sp_ref, x_ref, invd_ref, wa_ref, wb_ref, b_ref, o_ref):
    mean = sp_ref[...] * invd_ref[...]
    y = (jnp.dot(x_ref[...], wa_ref[...], preferred_element_type=jnp.float32)
         + jnp.dot(mean, wb_ref[...], preferred_element_type=jnp.float32)
         + b_ref[...])
    if relu:
        y = jnp.maximum(y, 0.0)
    o_ref[...] = y


_MAT_SPEC = pl.BlockSpec((128, 128), lambda i: (0, 0))
_VEC_SPEC = pl.BlockSpec((1, 128), lambda i: (0, 0))
_ROW_SPEC = pl.BlockSpec((R, 128), lambda i: (i, 0))


def _tc1(sp0, sp1, x, wa, wb, b):
    return pl.pallas_call(
        _tc1_body,
        grid=(N // R,),
        in_specs=[_ROW_SPEC, _ROW_SPEC, _ROW_SPEC, _MAT_SPEC, _MAT_SPEC, _VEC_SPEC],
        out_specs=[_ROW_SPEC, _ROW_SPEC],
        out_shape=[
            jax.ShapeDtypeStruct((N, 128), jnp.float32),
            jax.ShapeDtypeStruct((N, 128), jnp.float32),
        ],
        name="sage_tc1",
    )(sp0, sp1, x, wa, wb, b)


def _tc23(sp, x, invd, wa, wb, b, relu):
    return pl.pallas_call(
        functools.partial(_tc23_body, relu),
        grid=(N // R,),
        in_specs=[_ROW_SPEC, _ROW_SPEC, _ROW_SPEC, _MAT_SPEC, _MAT_SPEC, _VEC_SPEC],
        out_specs=_ROW_SPEC,
        out_shape=jax.ShapeDtypeStruct((N, 128), jnp.float32),
        name="sage_tc23",
    )(sp, x, invd, wa, wb, b)


def kernel(h, edge_index, W1, b1, W2, b2, W3, b3):
    f32 = jnp.float32
    src_r = edge_index[0].reshape(ER, CH)
    dst_r = edge_index[1].reshape(ER, CH)

    ones = jnp.ones((ZR, 128), f32)
    z = jnp.zeros((ZR, 128), f32)

    sp0, sp1 = _agg1(h, ones, src_r, dst_r, z)
    x1, invd = _tc1(sp0, sp1, h, W1[:128], W1[128:], b1.reshape(1, 128))
    s2 = _agg2(x1, src_r, dst_r, z)
    x2 = _tc23(s2, x1, invd, W2[:128], W2[128:], b2.reshape(1, 128), True)
    s3 = _agg2(x2, src_r, dst_r, z)
    x3 = _tc23(s3, x2, invd, W3[:128], W3[128:], b3.reshape(1, 128), False)
    return x3


# speedup vs baseline: 11.2396x; 1.1362x over previous
import functools

import jax
import jax.numpy as jnp
from jax import lax
from jax.experimental import pallas as pl
from jax.experimental.pallas import tpu as pltpu
from jax.experimental.pallas import tpu_sc as plsc

N = 10000
NC = 2
NS = 16
ZR = N // NS
E = 320000
CH = 128
ER = E // CH
CPT = ER // NS
KB = 39
NB = CPT // KB
DW = 16
HW = 64
R = 2000


def _edge_loop(src_hbm, dst_hbm, xtab, acc, sidxs, didxs,
               rows, gsems, ssems, semi, s, deg=None, idx_dbuf=True):
    base = s * CPT
    npre = 2 if idx_dbuf else 1

    for p in range(npre):
        pltpu.async_copy(src_hbm.at[pl.ds(base + p * KB, KB)], sidxs[p], semi)
        pltpu.async_copy(dst_hbm.at[pl.ds(base + p * KB, KB)], didxs[p], semi)

    def one_block(bb, p):
        off = pl.ds(base + bb * KB, KB)
        pltpu.make_async_copy(src_hbm.at[off], sidxs[p], semi).wait()
        pltpu.make_async_copy(dst_hbm.at[off], didxs[p], semi).wait()
        sidx = sidxs[p]
        didx = didxs[p]
        pltpu.async_copy(xtab.at[sidx.at[0]], rows[0], gsems[0])
        pltpu.async_copy(xtab.at[sidx.at[1]], rows[1], gsems[1])
        for ch in range(KB):
            j = ch % 3
            pltpu.make_async_copy(xtab.at[sidx.at[ch]], rows[j], gsems[j]).wait()
            pltpu.async_copy(rows[j], acc.at[didx.at[ch]], ssems[j], add=True)
            if deg is not None:
                c, orows, dacc = deg

                @pl.when((bb < NB // 2) == (c == 0))
                def _():
                    pltpu.sync_copy(orows, dacc.at[didx.at[ch]], add=True)
            if ch + 2 < KB:
                k = (ch + 2) % 3
                if ch >= 1:
                    pltpu.make_async_copy(rows[k], acc.at[didx.at[ch - 1]],
                                          ssems[k]).wait()
                pltpu.async_copy(xtab.at[sidx.at[ch + 2]], rows[k], gsems[k])
        for ch in (KB - 3, KB - 2, KB - 1):
            pltpu.make_async_copy(rows[ch % 3], acc.at[didx.at[ch]],
                                  ssems[ch % 3]).wait()

    if idx_dbuf:
        def blk2(b2, carry):
            for p in range(2):
                bb = b2 * 2 + p
                one_block(bb, p)

                @pl.when(bb + 2 < NB)
                def _():
                    off2 = pl.ds(base + (bb + 2) * KB, KB)
                    pltpu.async_copy(src_hbm.at[off2], sidxs[p], semi)
                    pltpu.async_copy(dst_hbm.at[off2], didxs[p], semi)
            return carry

        lax.fori_loop(0, NB // 2, blk2, 0)
    else:
        def blk1(b, carry):
            one_block(b, 0)

            @pl.when(b + 1 < NB)
            def _():
                off2 = pl.ds(base + (b + 1) * KB, KB)
                pltpu.async_copy(src_hbm.at[off2], sidxs[0], semi)
                pltpu.async_copy(dst_hbm.at[off2], didxs[0], semi)
            return carry

        lax.fori_loop(0, NB, blk1, 0)

    @pl.when(s < ER - NS * CPT)
    def _():
        pltpu.sync_copy(src_hbm.at[pl.ds(NS * CPT + s, 1)], sidxs[0].at[pl.ds(0, 1)])
        pltpu.sync_copy(dst_hbm.at[pl.ds(NS * CPT + s, 1)], didxs[0].at[pl.ds(0, 1)])
        pltpu.async_copy(xtab.at[sidxs[0].at[0]], rows[0], gsems[0]).wait()
        pltpu.sync_copy(rows[0], acc.at[didxs[0].at[0]], add=True)
        if deg is not None:
            c, orows, dacc = deg

            @pl.when(c == 0)
            def _():
                pltpu.sync_copy(orows, dacc.at[didxs[0].at[0]], add=True)


def _sc_agg1_body(x_hbm, ones_hbm, src_hbm, dst_hbm, z_hbm, out0_hbm, out1_hbm,
                  xtab, acc, dacc, orows, sidx0, didx0,
                  rows0, rows1, rows2, g0, g1, g2, s0, s1, s2, semi):
    c = lax.axis_index("c")
    s = lax.axis_index("s")
    rs = pl.ds(s * ZR, ZR)

    pltpu.sync_copy(x_hbm.at[rs, pl.ds(c * HW, HW)], xtab.at[rs])
    pltpu.sync_copy(z_hbm.at[:, pl.ds(0, HW)], acc.at[rs])
    pltpu.sync_copy(z_hbm.at[:, pl.ds(0, DW)], dacc.at[rs])
    pltpu.sync_copy(ones_hbm.at[pl.ds(0, CH), pl.ds(0, DW)], orows)
    plsc.subcore_barrier()

    _edge_loop(src_hbm, dst_hbm, xtab, acc, (sidx0,), (didx0,),
               (rows0, rows1, rows2), (g0, g1, g2), (s0, s1, s2), semi, s,
               deg=(c, orows, dacc), idx_dbuf=False)

    plsc.subcore_barrier()

    @pl.when(c == 0)
    def _():
        pltpu.sync_copy(acc.at[rs], out0_hbm.at[rs, pl.ds(0, HW)])
        pltpu.sync_copy(dacc.at[rs], out0_hbm.at[rs, pl.ds(HW, DW)])

    @pl.when(c == 1)
    def _():
        pltpu.sync_copy(acc.at[rs], out1_hbm.at[rs, pl.ds(0, HW)])
        pltpu.sync_copy(dacc.at[rs], out1_hbm.at[rs, pl.ds(HW, DW)])


def _sc_agg2_body(x_hbm, src_hbm, dst_hbm, z_hbm, out_hbm,
                  xtab, acc, sidx0, didx0, sidx1, didx1,
                  rows0, rows1, rows2, g0, g1, g2, s0, s1, s2, semi):
    c = lax.axis_index("c")
    s = lax.axis_index("s")
    rs = pl.ds(s * ZR, ZR)
    cs = pl.ds(c * HW, HW)

    pltpu.sync_copy(x_hbm.at[rs, cs], xtab.at[rs])
    pltpu.sync_copy(z_hbm.at[:, pl.ds(0, HW)], acc.at[rs])
    plsc.subcore_barrier()

    _edge_loop(src_hbm, dst_hbm, xtab, acc, (sidx0, sidx1), (didx0, didx1),
               (rows0, rows1, rows2), (g0, g1, g2), (s0, s1, s2), semi, s)

    plsc.subcore_barrier()
    pltpu.sync_copy(acc.at[rs], out_hbm.at[rs, cs])


_SC_MESH = plsc.VectorSubcoreMesh(core_axis_name="c", subcore_axis_name="s")
_SC_PARAMS = pltpu.CompilerParams(use_tc_tiling_on_sc=False)


def _rows_sems_scratch():
    return [
        pltpu.VMEM((CH, HW), jnp.float32),
        pltpu.VMEM((CH, HW), jnp.float32),
        pltpu.VMEM((CH, HW), jnp.float32),
        pltpu.SemaphoreType.DMA,
        pltpu.SemaphoreType.DMA,
        pltpu.SemaphoreType.DMA,
        pltpu.SemaphoreType.DMA,
        pltpu.SemaphoreType.DMA,
        pltpu.SemaphoreType.DMA,
        pltpu.SemaphoreType.DMA,
    ]


_agg1 = pl.kernel(
    _sc_agg1_body,
    out_type=[jax.ShapeDtypeStruct((N, 128), jnp.float32),
              jax.ShapeDtypeStruct((N, 128), jnp.float32)],
    mesh=_SC_MESH,
    scratch_types=[
        pltpu.VMEM_SHARED((N, HW), jnp.float32),
        pltpu.VMEM_SHARED((N, HW), jnp.float32),
        pltpu.VMEM_SHARED((N, DW), jnp.float32),
        pltpu.VMEM((CH, DW), jnp.float32),
        pltpu.VMEM((KB, CH), jnp.int32),
        pltpu.VMEM((KB, CH), jnp.int32),
    ] + _rows_sems_scratch(),
    compiler_params=_SC_PARAMS,
    name="sage_sc_agg1",
)

_agg2 = pl.kernel(
    _sc_agg2_body,
    out_type=jax.ShapeDtypeStruct((N, 128), jnp.float32),
    mesh=_SC_MESH,
    scratch_types=[
        pltpu.VMEM_SHARED((N, HW), jnp.float32),
        pltpu.VMEM_SHARED((N, HW), jnp.float32),
        pltpu.VMEM((KB, CH), jnp.int32),
        pltpu.VMEM((KB, CH), jnp.int32),
        pltpu.VMEM((KB, CH), jnp.int32),
        pltpu.VMEM((KB, CH), jnp.int32),
    ] + _rows_sems_scratch(),
    compiler_params=_SC_PARAMS,
    name="sage_sc_agg2",
)


def _tc1_body(sp0_ref, sp1_ref, x_ref, wa_ref, wb_ref, b_ref, o_ref, invd_ref):
    v0 = sp0_ref[...]
    v1 = sp1_ref[...]
    ssum = jnp.concatenate([v0[:, :HW], v1[:, :HW]], axis=1)
    deg = v0[:, HW:HW + 1] + v1[:, HW:HW + 1]
    invd = 1.0 / jnp.maximum(deg, 1.0)
    mean = ssum * invd
    y = (jnp.dot(x_ref[...], wa_ref[...], preferred_element_type=jnp.float32)
         + jnp.dot(mean, wb_ref[...], preferred_element_type=jnp.float32)
         + b_ref[...])
    o_ref[...] = jnp.maximum(y, 0.0)
    invd_ref[...] = jnp.broadcast_to(invd, (invd.shape[0], 128))


def _tc23_body(relu, sp_ref, x_ref, invd_ref, wa_ref, wb_ref, b_ref, o_ref):
    mean = sp_ref[...] * invd_ref[...]
    y = (jnp.dot(x_ref[...], wa_ref[...], preferred_element_type=jnp.float32)
         + jnp.dot(mean, wb_ref[...], preferred_element_type=jnp.float32)
         + b_ref[...])
    if relu:
        y = jnp.maximum(y, 0.0)
    o_ref[...] = y


_MAT_SPEC = pl.BlockSpec((128, 128), lambda i: (0, 0))
_VEC_SPEC = pl.BlockSpec((1, 128), lambda i: (0, 0))
_ROW_SPEC = pl.BlockSpec((R, 128), lambda i: (i, 0))


def _tc1(sp0, sp1, x, wa, wb, b):
    return pl.pallas_call(
        _tc1_body,
        grid=(N // R,),
        in_specs=[_ROW_SPEC, _ROW_SPEC, _ROW_SPEC, _MAT_SPEC, _MAT_SPEC, _VEC_SPEC],
        out_specs=[_ROW_SPEC, _ROW_SPEC],
        out_shape=[
            jax.ShapeDtypeStruct((N, 128), jnp.float32),
            jax.ShapeDtypeStruct((N, 128), jnp.float32),
        ],
        name="sage_tc1",
    )(sp0, sp1, x, wa, wb, b)


def _tc23(sp, x, invd, wa, wb, b, relu):
    return pl.pallas_call(
        functools.partial(_tc23_body, relu),
        grid=(N // R,),
        in_specs=[_ROW_SPEC, _ROW_SPEC, _ROW_SPEC, _MAT_SPEC, _MAT_SPEC, _VEC_SPEC],
        out_specs=_ROW_SPEC,
        out_shape=jax.ShapeDtypeStruct((N, 128), jnp.float32),
        name="sage_tc23",
    )(sp, x, invd, wa, wb, b)


def kernel(h, edge_index, W1, b1, W2, b2, W3, b3):
    f32 = jnp.float32
    src_r = edge_index[0].reshape(ER, CH)
    dst_r = edge_index[1].reshape(ER, CH)

    ones = jnp.ones((ZR, 128), f32)
    z = jnp.zeros((ZR, 128), f32)

    sp0, sp1 = _agg1(h, ones, src_r, dst_r, z)
    x1, invd = _tc1(sp0, sp1, h, W1[:128], W1[128:], b1.reshape(1, 128))
    s2 = _agg2(x1, src_r, dst_r, z)
    x2 = _tc23(s2, x1, invd, W2[:128], W2[128:], b2.reshape(1, 128), True)
    s3 = _agg2(x2, src_r, dst_r, z)
    x3 = _tc23(s3, x2, invd, W3[:128], W3[128:], b3.reshape(1, 128), False)
    return x3
